# Initial kernel scaffold; baseline (speedup 1.0000x reference)
import jax
import jax.numpy as jnp
from jax.experimental import pallas as pl


def kernel(inputs, edge_index0, edge_index1, emb, W1, b1, W2, b2):
    raise NotImplementedError("write your pallas kernel here")


# speedup vs baseline: 5.9952x; 5.9952x over previous
import functools

import jax
import jax.numpy as jnp
from jax import lax
from jax.experimental import pallas as pl
from jax.experimental.pallas import tpu as pltpu
from jax.experimental.pallas import tpu_sc as plsc

_N = 10000
_NPAD = 10240
_E = 160000
_EPAD = 163840
_V = 100000
_H = 256
_HH = 128
_C = 128
_NC = 2
_NS = 16
_NW = _NC * _NS
_RPW = _NPAD // _NW
_ZROWS = _NPAD // _NS

_sc_mesh = functools.partial(
    plsc.VectorSubcoreMesh, core_axis_name="c", subcore_axis_name="s"
)


@functools.partial(
    pl.kernel,
    out_type=(
        jax.ShapeDtypeStruct((_NPAD, _H), jnp.float32),
        jax.ShapeDtypeStruct((_NS, 4, _NPAD), jnp.float32),
    ),
    mesh=_sc_mesh(),
    scratch_types=[
        pltpu.VMEM((4, 80), jnp.int32),
        pltpu.VMEM((80, _H), jnp.float32),
        pltpu.VMEM((640, 16), jnp.int32),
        pltpu.VMEM((_NPAD,), jnp.float32),
        pltpu.VMEM((_NPAD,), jnp.float32),
        pltpu.SemaphoreType.DMA,
    ],
    compiler_params=pltpu.CompilerParams(needs_layout_passes=False),
)
def _gather_hist(emb, ids, ehist, x_out, degp, ids_v, rows_v, eidx_v,
                 hist_a, hist_b, sem):
    c = lax.axis_index("c")
    s = lax.axis_index("s")
    wid = c * _NS + s

    pltpu.sync_copy(ids.at[wid], ids_v)
    for j in range(4):
        pltpu.async_copy(emb.at[ids_v.at[j]], rows_v, sem).wait()
        pltpu.sync_copy(rows_v, x_out.at[pl.ds(wid * _RPW + j * 80, 80)])

    ones = jnp.ones((16,), jnp.float32)
    zeros = jnp.zeros((16,), jnp.float32)
    for a, href in ((0, hist_a), (1, hist_b)):
        @pl.loop(0, _NPAD // 16)
        def _zero(i):
            href[pl.ds(i * 16, 16)] = zeros

        pltpu.sync_copy(ehist.at[c, a, s], eidx_v)

        @pl.loop(0, 640)
        def _accum(i):
            plsc.addupdate_scatter(href, [eidx_v[i]], ones)

        pltpu.sync_copy(href, degp.at[s, c * 2 + a])


def _make_spmm(nb, table_rows, split_by_core):
    @functools.partial(
        pl.kernel,
        out_type=jax.ShapeDtypeStruct((_NC, _NPAD, _HH), jnp.float32),
        mesh=_sc_mesh(),
        scratch_types=[
            pltpu.VMEM((nb, 128), jnp.int32),
            pltpu.VMEM((nb, 128), jnp.int32),
            pltpu.VMEM((128, _HH), jnp.float32),
            pltpu.VMEM_SHARED((_NPAD, _HH), jnp.float32),
            pltpu.SemaphoreType.DMA,
        ],
        compiler_params=pltpu.CompilerParams(needs_layout_passes=False),
    )
    def spmm(table, srci, dsti, out, idxs_v, idxd_v, buf, acc, sem):
        c = lax.axis_index("c")
        s = lax.axis_index("s")

        if split_by_core:
            pltpu.sync_copy(srci.at[c, s], idxs_v)
            pltpu.sync_copy(dsti.at[s], idxd_v)
        else:
            wid = c * _NS + s
            pltpu.sync_copy(srci.at[wid], idxs_v)
            pltpu.sync_copy(dsti.at[wid], idxd_v)

        zeros = jnp.zeros((16,), jnp.float32)

        @pl.loop(0, 128)
        def _zrow(i):
            for j in range(_HH // 16):
                buf[i, pl.ds(j * 16, 16)] = zeros

        for t in range(_ZROWS // 128):
            pltpu.sync_copy(buf, acc.at[pl.ds(s * _ZROWS + t * 128, 128)])
        plsc.subcore_barrier()

        @pl.loop(0, nb)
        def _edge(j):
            pltpu.async_copy(table.at[idxs_v.at[j]], buf, sem).wait()
            pltpu.sync_copy(buf, acc.at[idxd_v.at[j]], add=True)

        plsc.subcore_barrier()
        pltpu.sync_copy(acc.at[pl.ds(s * _ZROWS, _ZROWS)],
                        out.at[c, pl.ds(s * _ZROWS, _ZROWS)])

    return spmm


_spmm_l1 = _make_spmm(_EPAD // _NS // 128, _NC * _NPAD, True)
_spmm_l2 = _make_spmm(_EPAD // _NW // 128, _NPAD, False)


def _norms_body(degp_ref, norms_ref):
    d = degp_ref[0]
    for s in range(1, _NS):
        d = d + degp_ref[s]
    norms_ref[...] = lax.rsqrt(jnp.clip(d, 1.0, None))


def _scale_body(x_ref, nt_ref, s1_ref):
    t = x_ref[...] * nt_ref[:, 0:1]
    s1_ref[0] = t[:, :_HH]
    s1_ref[1] = t[:, _HH:]


def _mlp_body(a_ref, nt_ref, w1_ref, b1_ref, w2_ref, z_ref):
    h = jnp.concatenate([a_ref[0], a_ref[1]], axis=1) * nt_ref[:, 1:2]
    h = jnp.dot(h, w1_ref[...], preferred_element_type=jnp.float32)
    h = jnp.maximum(h + b1_ref[...], 0.0) * nt_ref[:, 2:3]
    z_ref[...] = jnp.dot(h, w2_ref[...], preferred_element_type=jnp.float32)


def _fin_body(a_ref, nt_ref, b2_ref, o_ref):
    o_ref[...] = (a_ref[0] + a_ref[1]) * nt_ref[:, 3:4] + b2_ref[...]


_BM = 1024


def _tc_norms(degp):
    return pl.pallas_call(
        _norms_body,
        out_shape=jax.ShapeDtypeStruct((4, _NPAD), jnp.float32),
    )(degp)


def _tc_scale(x, norms_t):
    return pl.pallas_call(
        _scale_body,
        grid=(_NPAD // _BM,),
        in_specs=[
            pl.BlockSpec((_BM, _H), lambda i: (i, 0)),
            pl.BlockSpec((_BM, 4), lambda i: (i, 0)),
        ],
        out_specs=pl.BlockSpec((_NC, _BM, _HH), lambda i: (0, i, 0)),
        out_shape=jax.ShapeDtypeStruct((_NC, _NPAD, _HH), jnp.float32),
    )(x, norms_t)


def _tc_mlp(agg0, norms_t, w1, b1, w2):
    return pl.pallas_call(
        _mlp_body,
        grid=(_NPAD // _BM,),
        in_specs=[
            pl.BlockSpec((_NC, _BM, _HH), lambda i: (0, i, 0)),
            pl.BlockSpec((_BM, 4), lambda i: (i, 0)),
            pl.BlockSpec((_H, _H), lambda i: (0, 0)),
            pl.BlockSpec((1, _H), lambda i: (0, 0)),
            pl.BlockSpec((_H, _C), lambda i: (0, 0)),
        ],
        out_specs=pl.BlockSpec((_BM, _C), lambda i: (i, 0)),
        out_shape=jax.ShapeDtypeStruct((_NPAD, _C), jnp.float32),
    )(agg0, norms_t, w1, b1, w2)


def _tc_fin(agg1, norms_t, b2):
    return pl.pallas_call(
        _fin_body,
        grid=(_NPAD // _BM,),
        in_specs=[
            pl.BlockSpec((_NC, _BM, _C), lambda i: (0, i, 0)),
            pl.BlockSpec((_BM, 4), lambda i: (i, 0)),
            pl.BlockSpec((1, _C), lambda i: (0, 0)),
        ],
        out_specs=pl.BlockSpec((_BM, _C), lambda i: (i, 0)),
        out_shape=jax.ShapeDtypeStruct((_NPAD, _C), jnp.float32),
    )(agg1, norms_t, b2)


@jax.jit
def kernel(inputs, edge_index0, edge_index1, emb, W1, b1, W2, b2):
    padfill = _N + (jnp.arange(_EPAD - _E, dtype=jnp.int32) % (_NPAD - _N))

    def pad_e(v):
        return jnp.concatenate([v, padfill])

    src0 = pad_e(edge_index0[0])
    dst0 = pad_e(edge_index0[1])
    src1 = pad_e(edge_index1[0])
    dst1 = pad_e(edge_index1[1])

    ids = jnp.concatenate(
        [inputs, jnp.zeros((_NPAD - _N,), jnp.int32)]
    ).reshape(_NW, 4, 80)
    ehist = jnp.stack([src0, dst0, src1, dst1]).reshape(2, 2, _NS, 640, 16)

    x, degp = _gather_hist(emb, ids, ehist)
    norms_t = _tc_norms(degp).T

    s1 = _tc_scale(x, norms_t)
    src_a = jnp.stack([src0, src0 + _NPAD]).reshape(_NC, _NS, 80, 128)
    dst_a = dst0.reshape(_NS, 80, 128)
    agg0 = _spmm_l1(s1.reshape(_NC * _NPAD, _HH), src_a, dst_a)

    z2 = _tc_mlp(agg0, norms_t, W1, b1.reshape(1, _H), W2)

    src_b = src1.reshape(_NW, 40, 128)
    dst_b = dst1.reshape(_NW, 40, 128)
    agg1 = _spmm_l2(z2, src_b, dst_b)

    out = _tc_fin(agg1, norms_t, b2.reshape(1, _C))
    return out[:_N]


# speedup vs baseline: 6.4834x; 1.0814x over previous
import functools

import jax
import jax.numpy as jnp
from jax import lax
from jax.experimental import pallas as pl
from jax.experimental.pallas import tpu as pltpu
from jax.experimental.pallas import tpu_sc as plsc

_N = 10000
_NPAD = 10240
_E = 160000
_EPAD = 163840
_V = 100000
_H = 256
_HH = 128
_C = 128
_NC = 2
_NS = 16
_NW = _NC * _NS
_RPW = _NPAD // _NW
_ZROWS = _NPAD // _NS

_sc_mesh = functools.partial(
    plsc.VectorSubcoreMesh, core_axis_name="c", subcore_axis_name="s"
)


@functools.partial(
    pl.kernel,
    out_type=(
        jax.ShapeDtypeStruct((_NPAD, _H), jnp.float32),
        jax.ShapeDtypeStruct((_NS, 4, _NPAD), jnp.float32),
    ),
    mesh=_sc_mesh(),
    scratch_types=[
        pltpu.VMEM((4, 80), jnp.int32),
        pltpu.VMEM((4, 80, _H), jnp.float32),
        pltpu.VMEM((80, 128), jnp.int32),
        pltpu.VMEM((_NPAD,), jnp.float32),
        pltpu.VMEM((_NPAD,), jnp.float32),
        pltpu.SemaphoreType.DMA,
        pltpu.SemaphoreType.DMA,
        pltpu.SemaphoreType.DMA,
        pltpu.SemaphoreType.DMA,
        pltpu.SemaphoreType.DMA,
    ],
    compiler_params=pltpu.CompilerParams(needs_layout_passes=False),
)
def _gather_hist(emb, ids, ehist, x_out, degp, ids_v, rows_v, eidx_v,
                 hist_a, hist_b, sg0, sg1, sg2, sg3, sw):
    c = lax.axis_index("c")
    s = lax.axis_index("s")
    wid = c * _NS + s
    gsem = (sg0, sg1, sg2, sg3)

    pltpu.sync_copy(ids.at[wid], ids_v)
    dg = [
        pltpu.async_copy(emb.at[ids_v.at[j]], rows_v.at[j], gsem[j])
        for j in range(4)
    ]

    ones = jnp.ones((16,), jnp.float32)
    zeros = jnp.zeros((16,), jnp.float32)
    for a, href in ((0, hist_a), (1, hist_b)):
        @pl.loop(0, _NPAD // 16)
        def _zero(i):
            href[pl.ds(i * 16, 16)] = zeros

        pltpu.sync_copy(ehist.at[c, a, s], eidx_v)

        @pl.loop(0, 80)
        def _accum(r):
            for jj in range(8):
                plsc.addupdate_scatter(
                    href, [eidx_v[r, pl.ds(jj * 16, 16)]], ones)

        pltpu.sync_copy(href, degp.at[s, c * 2 + a])

    dw = []
    for j in range(4):
        dg[j].wait()
        dw.append(
            pltpu.async_copy(
                rows_v.at[j], x_out.at[pl.ds(wid * _RPW + j * 80, 80)], sw
            )
        )
    for d in dw:
        d.wait()


def _make_spmm(nb, k, table_rows, split_by_core, pipelined):
    assert nb % 2 == 0

    @functools.partial(
        pl.kernel,
        out_type=jax.ShapeDtypeStruct((_NC, _NPAD, _HH), jnp.float32),
        mesh=_sc_mesh(),
        scratch_types=[
            pltpu.VMEM((nb, k), jnp.int32),
            pltpu.VMEM((nb, k), jnp.int32),
            pltpu.VMEM((k, _HH), jnp.float32),
            pltpu.VMEM((k, _HH), jnp.float32),
            pltpu.VMEM_SHARED((_NPAD, _HH), jnp.float32),
            pltpu.SemaphoreType.DMA,
            pltpu.SemaphoreType.DMA,
        ],
        compiler_params=pltpu.CompilerParams(needs_layout_passes=False),
    )
    def spmm(table, srci, dsti, out, idxs_v, idxd_v, b0, b1, acc,
             g0, g1):
        c = lax.axis_index("c")
        s = lax.axis_index("s")

        if split_by_core:
            pltpu.sync_copy(srci.at[c, s], idxs_v)
            pltpu.sync_copy(dsti.at[s], idxd_v)
        else:
            wid = c * _NS + s
            pltpu.sync_copy(srci.at[wid], idxs_v)
            pltpu.sync_copy(dsti.at[wid], idxd_v)

        zeros = jnp.zeros((16,), jnp.float32)

        @pl.loop(0, k)
        def _zrow(i):
            for j in range(_HH // 16):
                b0[i, pl.ds(j * 16, 16)] = zeros

        for t in range(_ZROWS // k):
            pltpu.sync_copy(b0, acc.at[pl.ds(s * _ZROWS + t * k, k)])
        plsc.subcore_barrier()

        if pipelined:
            def g_start(j, bf, gs):
                pltpu.async_copy(table.at[idxs_v.at[j]], bf, gs)

            def g_wait(j, bf, gs):
                pltpu.make_async_copy(table.at[idxs_v.at[j]], bf, gs).wait()

            g_start(0, b0, g0)

            @pl.loop(0, nb // 2)
            def _edge(jj):
                j = jj * 2
                g_wait(j, b0, g0)
                g_start(j + 1, b1, g1)
                pltpu.sync_copy(b0, acc.at[idxd_v.at[j]], add=True)
                g_wait(j + 1, b1, g1)
                g_start(jnp.minimum(j + 2, nb - 1), b0, g0)
                pltpu.sync_copy(b1, acc.at[idxd_v.at[j + 1]], add=True)

            g_wait(nb - 1, b0, g0)
        else:
            @pl.loop(0, nb)
            def _edge(j):
                pltpu.async_copy(table.at[idxs_v.at[j]], b0, g0).wait()
                pltpu.sync_copy(b0, acc.at[idxd_v.at[j]], add=True)

        plsc.subcore_barrier()
        pltpu.sync_copy(acc.at[pl.ds(s * _ZROWS, _ZROWS)],
                        out.at[c, pl.ds(s * _ZROWS, _ZROWS)])

    return spmm


_K1 = 128
_K2 = 128
_spmm_l1 = _make_spmm(_EPAD // _NS // _K1, _K1, _NC * _NPAD, True, False)
_spmm_l2 = _make_spmm(_EPAD // _NW // _K2, _K2, _NPAD, False, False)


_BM = 1024


def _scale_body(x_ref, degp_ref, s1_ref, nt_ref):
    d = degp_ref[0]
    for i in range(1, _NS):
        d = d + degp_ref[i]
    nt = lax.rsqrt(jnp.clip(d, 1.0, None)).T
    nt_ref[...] = nt
    t = x_ref[...] * nt[:, 0:1]
    s1_ref[0] = t[:, :_HH]
    s1_ref[1] = t[:, _HH:]


def _mlp_body(a_ref, nt_ref, w1_ref, b1_ref, w2_ref, z_ref):
    h = jnp.concatenate([a_ref[0], a_ref[1]], axis=1) * nt_ref[:, 1:2]
    h = jnp.dot(h, w1_ref[...], preferred_element_type=jnp.float32)
    h = jnp.maximum(h + b1_ref[...], 0.0) * nt_ref[:, 2:3]
    z_ref[...] = jnp.dot(h, w2_ref[...], preferred_element_type=jnp.float32)


def _fin_body(a_ref, nt_ref, b2_ref, o_ref):
    o_ref[...] = (a_ref[0] + a_ref[1]) * nt_ref[:, 3:4] + b2_ref[...]


def _tc_scale(x, degp):
    return pl.pallas_call(
        _scale_body,
        grid=(_NPAD // _BM,),
        in_specs=[
            pl.BlockSpec((_BM, _H), lambda i: (i, 0)),
            pl.BlockSpec((_NS, 4, _BM), lambda i: (0, 0, i)),
        ],
        out_specs=[
            pl.BlockSpec((_NC, _BM, _HH), lambda i: (0, i, 0)),
            pl.BlockSpec((_BM, 4), lambda i: (i, 0)),
        ],
        out_shape=[
            jax.ShapeDtypeStruct((_NC, _NPAD, _HH), jnp.float32),
            jax.ShapeDtypeStruct((_NPAD, 4), jnp.float32),
        ],
    )(x, degp)


def _tc_mlp(agg0, norms_t, w1, b1, w2):
    return pl.pallas_call(
        _mlp_body,
        grid=(_NPAD // _BM,),
        in_specs=[
            pl.BlockSpec((_NC, _BM, _HH), lambda i: (0, i, 0)),
            pl.BlockSpec((_BM, 4), lambda i: (i, 0)),
            pl.BlockSpec((_H, _H), lambda i: (0, 0)),
            pl.BlockSpec((1, _H), lambda i: (0, 0)),
            pl.BlockSpec((_H, _C), lambda i: (0, 0)),
        ],
        out_specs=pl.BlockSpec((_BM, _C), lambda i: (i, 0)),
        out_shape=jax.ShapeDtypeStruct((_NPAD, _C), jnp.float32),
    )(agg0, norms_t, w1, b1, w2)


def _tc_fin(agg1, norms_t, b2):
    return pl.pallas_call(
        _fin_body,
        grid=(_NPAD // _BM,),
        in_specs=[
            pl.BlockSpec((_NC, _BM, _C), lambda i: (0, i, 0)),
            pl.BlockSpec((_BM, 4), lambda i: (i, 0)),
            pl.BlockSpec((1, _C), lambda i: (0, 0)),
        ],
        out_specs=pl.BlockSpec((_BM, _C), lambda i: (i, 0)),
        out_shape=jax.ShapeDtypeStruct((_NPAD, _C), jnp.float32),
    )(agg1, norms_t, b2)


@jax.jit
def kernel(inputs, edge_index0, edge_index1, emb, W1, b1, W2, b2):
    padfill = _N + (jnp.arange(_EPAD - _E, dtype=jnp.int32) % (_NPAD - _N))

    def pad_e(v):
        return jnp.concatenate([v, padfill])

    src0 = pad_e(edge_index0[0])
    dst0 = pad_e(edge_index0[1])
    src1 = pad_e(edge_index1[0])
    dst1 = pad_e(edge_index1[1])

    ids = jnp.concatenate(
        [inputs, jnp.zeros((_NPAD - _N,), jnp.int32)]
    ).reshape(_NW, 4, 80)
    ehist = jnp.stack([src0, dst0, src1, dst1]).reshape(2, 2, _NS, 80, 128)

    x, degp = _gather_hist(emb, ids, ehist)
    s1, norms_t = _tc_scale(x, degp)

    src_a = jnp.stack([src0, src0 + _NPAD]).reshape(_NC, _NS, -1, _K1)
    dst_a = dst0.reshape(_NS, -1, _K1)
    agg0 = _spmm_l1(s1.reshape(_NC * _NPAD, _HH), src_a, dst_a)

    z2 = _tc_mlp(agg0, norms_t, W1, b1.reshape(1, _H), W2)

    src_b = src1.reshape(_NW, -1, _K2)
    dst_b = dst1.reshape(_NW, -1, _K2)
    agg1 = _spmm_l2(z2, src_b, dst_b)

    out = _tc_fin(agg1, norms_t, b2.reshape(1, _C))
    return out[:_N]


# speedup vs baseline: 7.8823x; 1.2158x over previous
import functools

import jax
import jax.numpy as jnp
from jax import lax
from jax.experimental import pallas as pl
from jax.experimental.pallas import tpu as pltpu
from jax.experimental.pallas import tpu_sc as plsc

_N = 10000
_NPAD = 10240
_E = 160000
_EPAD = 163840
_V = 100000
_H = 256
_HH = 128
_C = 128
_NC = 2
_NS = 16
_NW = _NC * _NS
_RPW = _NPAD // _NW
_ZROWS = _NPAD // _NS

_sc_mesh = functools.partial(
    plsc.VectorSubcoreMesh, core_axis_name="c", subcore_axis_name="s"
)


@functools.partial(
    pl.kernel,
    out_type=(
        jax.ShapeDtypeStruct((_NPAD, _H), jnp.float32),
        jax.ShapeDtypeStruct((_NS, 4, _NPAD), jnp.float32),
    ),
    mesh=_sc_mesh(),
    scratch_types=[
        pltpu.VMEM((4, 80), jnp.int32),
        pltpu.VMEM((4, 80, _H), jnp.float32),
        pltpu.VMEM((80, 128), jnp.int32),
        pltpu.VMEM((_NPAD,), jnp.float32),
        pltpu.VMEM((_NPAD,), jnp.float32),
        pltpu.SemaphoreType.DMA,
        pltpu.SemaphoreType.DMA,
        pltpu.SemaphoreType.DMA,
        pltpu.SemaphoreType.DMA,
        pltpu.SemaphoreType.DMA,
    ],
    compiler_params=pltpu.CompilerParams(needs_layout_passes=False),
)
def _gather_hist(emb, ids, ehist, x_out, degp, ids_v, rows_v, eidx_v,
                 hist_a, hist_b, sg0, sg1, sg2, sg3, sw):
    c = lax.axis_index("c")
    s = lax.axis_index("s")
    wid = c * _NS + s
    gsem = (sg0, sg1, sg2, sg3)

    pltpu.sync_copy(ids.at[wid], ids_v)
    dg = [
        pltpu.async_copy(emb.at[ids_v.at[j]], rows_v.at[j], gsem[j])
        for j in range(4)
    ]

    ones = jnp.ones((16,), jnp.float32)
    zeros = jnp.zeros((16,), jnp.float32)
    for a, href in ((0, hist_a), (1, hist_b)):
        @pl.loop(0, _NPAD // 16)
        def _zero(i):
            href[pl.ds(i * 16, 16)] = zeros

        pltpu.sync_copy(ehist.at[c, a, s], eidx_v)

        @pl.loop(0, 80)
        def _accum(r):
            for jj in range(8):
                plsc.addupdate_scatter(
                    href, [eidx_v[r, pl.ds(jj * 16, 16)]], ones)

        pltpu.sync_copy(href, degp.at[s, c * 2 + a])

    dw = []
    for j in range(4):
        dg[j].wait()
        dw.append(
            pltpu.async_copy(
                rows_v.at[j], x_out.at[pl.ds(wid * _RPW + j * 80, 80)], sw
            )
        )
    for d in dw:
        d.wait()


def _make_spmm(nb, k, table_rows, split_by_core, nseg):
    assert nb % (2 * nseg) == 0
    seg_nb = nb // nseg
    assert seg_nb % 8 == 0

    @functools.partial(
        pl.kernel,
        out_type=jax.ShapeDtypeStruct((_NC, _NPAD, _HH), jnp.float32),
        mesh=_sc_mesh(),
        scratch_types=[
            pltpu.VMEM((seg_nb, k), jnp.int32),
            pltpu.VMEM((seg_nb, k), jnp.int32),
            pltpu.VMEM((k, _HH), jnp.float32),
            pltpu.VMEM((k, _HH), jnp.float32),
            pltpu.VMEM_SHARED((_NPAD, _HH), jnp.float32),
            pltpu.SemaphoreType.DMA,
            pltpu.SemaphoreType.DMA,
        ],
        compiler_params=pltpu.CompilerParams(needs_layout_passes=False),
    )
    def spmm(table, srci, dsti, out, idxs_v, idxd_v, b0, b1, acc,
             g0, g1):
        c = lax.axis_index("c")
        s = lax.axis_index("s")

        zeros = jnp.zeros((16,), jnp.float32)

        @pl.loop(0, k)
        def _zrow(i):
            for j in range(_HH // 16):
                b0[i, pl.ds(j * 16, 16)] = zeros

        for t in range(_ZROWS // k):
            pltpu.sync_copy(b0, acc.at[pl.ds(s * _ZROWS + t * k, k)])
        plsc.subcore_barrier()

        def g_start(j, bf, gs):
            pltpu.async_copy(table.at[idxs_v.at[j]], bf, gs)

        def g_wait(j, bf, gs):
            pltpu.make_async_copy(table.at[idxs_v.at[j]], bf, gs).wait()

        for seg in range(nseg):
            lo = seg * seg_nb
            if split_by_core:
                pltpu.sync_copy(srci.at[c, s, pl.ds(lo, seg_nb)], idxs_v)
                pltpu.sync_copy(dsti.at[s, pl.ds(lo, seg_nb)], idxd_v)
            else:
                wid = c * _NS + s
                pltpu.sync_copy(srci.at[wid, pl.ds(lo, seg_nb)], idxs_v)
                pltpu.sync_copy(dsti.at[wid, pl.ds(lo, seg_nb)], idxd_v)

            g_start(0, b0, g0)

            @pl.loop(0, seg_nb // 2)
            def _edge(jj):
                j = jj * 2
                g_wait(j, b0, g0)
                g_start(j + 1, b1, g1)
                pltpu.sync_copy(b0, acc.at[idxd_v.at[j]], add=True)
                g_wait(j + 1, b1, g1)
                g_start(jnp.minimum(j + 2, seg_nb - 1), b0, g0)
                pltpu.sync_copy(b1, acc.at[idxd_v.at[j + 1]], add=True)

            g_wait(seg_nb - 1, b0, g0)

        plsc.subcore_barrier()
        pltpu.sync_copy(acc.at[pl.ds(s * _ZROWS, _ZROWS)],
                        out.at[c, pl.ds(s * _ZROWS, _ZROWS)])

    return spmm


_K1 = 128
_K2 = 128
_spmm_l1 = _make_spmm(_EPAD // _NS // _K1, _K1, _NC * _NPAD, True, 2)
_spmm_l2 = _make_spmm(_EPAD // _NW // _K2, _K2, _NPAD, False, 1)


_BM = 1024


def _scale_body(x_ref, degp_ref, s1_ref, nt_ref):
    d = degp_ref[0]
    for i in range(1, _NS):
        d = d + degp_ref[i]
    nt = lax.rsqrt(jnp.clip(d, 1.0, None)).T
    nt_ref[...] = nt
    t = x_ref[...] * nt[:, 0:1]
    s1_ref[0] = t[:, :_HH]
    s1_ref[1] = t[:, _HH:]


def _mlp_body(a_ref, nt_ref, w1_ref, b1_ref, w2_ref, z_ref):
    h = jnp.concatenate([a_ref[0], a_ref[1]], axis=1) * nt_ref[:, 1:2]
    h = jnp.dot(h, w1_ref[...], preferred_element_type=jnp.float32)
    h = jnp.maximum(h + b1_ref[...], 0.0) * nt_ref[:, 2:3]
    z_ref[...] = jnp.dot(h, w2_ref[...], preferred_element_type=jnp.float32)


def _fin_body(a_ref, nt_ref, b2_ref, o_ref):
    o_ref[...] = (a_ref[0] + a_ref[1]) * nt_ref[:, 3:4] + b2_ref[...]


def _tc_scale(x, degp):
    return pl.pallas_call(
        _scale_body,
        grid=(_NPAD // _BM,),
        in_specs=[
            pl.BlockSpec((_BM, _H), lambda i: (i, 0)),
            pl.BlockSpec((_NS, 4, _BM), lambda i: (0, 0, i)),
        ],
        out_specs=[
            pl.BlockSpec((_NC, _BM, _HH), lambda i: (0, i, 0)),
            pl.BlockSpec((_BM, 4), lambda i: (i, 0)),
        ],
        out_shape=[
            jax.ShapeDtypeStruct((_NC, _NPAD, _HH), jnp.float32),
            jax.ShapeDtypeStruct((_NPAD, 4), jnp.float32),
        ],
    )(x, degp)


def _tc_mlp(agg0, norms_t, w1, b1, w2):
    return pl.pallas_call(
        _mlp_body,
        grid=(_NPAD // _BM,),
        in_specs=[
            pl.BlockSpec((_NC, _BM, _HH), lambda i: (0, i, 0)),
            pl.BlockSpec((_BM, 4), lambda i: (i, 0)),
            pl.BlockSpec((_H, _H), lambda i: (0, 0)),
            pl.BlockSpec((1, _H), lambda i: (0, 0)),
            pl.BlockSpec((_H, _C), lambda i: (0, 0)),
        ],
        out_specs=pl.BlockSpec((_BM, _C), lambda i: (i, 0)),
        out_shape=jax.ShapeDtypeStruct((_NPAD, _C), jnp.float32),
    )(agg0, norms_t, w1, b1, w2)


def _tc_fin(agg1, norms_t, b2):
    return pl.pallas_call(
        _fin_body,
        grid=(_NPAD // _BM,),
        in_specs=[
            pl.BlockSpec((_NC, _BM, _C), lambda i: (0, i, 0)),
            pl.BlockSpec((_BM, 4), lambda i: (i, 0)),
            pl.BlockSpec((1, _C), lambda i: (0, 0)),
        ],
        out_specs=pl.BlockSpec((_BM, _C), lambda i: (i, 0)),
        out_shape=jax.ShapeDtypeStruct((_NPAD, _C), jnp.float32),
    )(agg1, norms_t, b2)


@jax.jit
def kernel(inputs, edge_index0, edge_index1, emb, W1, b1, W2, b2):
    padfill = _N + (jnp.arange(_EPAD - _E, dtype=jnp.int32) % (_NPAD - _N))

    def pad_e(v):
        return jnp.concatenate([v, padfill])

    src0 = pad_e(edge_index0[0])
    dst0 = pad_e(edge_index0[1])
    src1 = pad_e(edge_index1[0])
    dst1 = pad_e(edge_index1[1])

    ids = jnp.concatenate(
        [inputs, jnp.zeros((_NPAD - _N,), jnp.int32)]
    ).reshape(_NW, 4, 80)
    ehist = jnp.stack([src0, dst0, src1, dst1]).reshape(2, 2, _NS, 80, 128)

    x, degp = _gather_hist(emb, ids, ehist)
    s1, norms_t = _tc_scale(x, degp)

    src_a = jnp.stack([src0, src0 + _NPAD]).reshape(_NC, _NS, -1, _K1)
    dst_a = dst0.reshape(_NS, -1, _K1)
    agg0 = _spmm_l1(s1.reshape(_NC * _NPAD, _HH), src_a, dst_a)

    z2 = _tc_mlp(agg0, norms_t, W1, b1.reshape(1, _H), W2)

    src_b = src1.reshape(_NW, -1, _K2)
    dst_b = dst1.reshape(_NW, -1, _K2)
    agg1 = _spmm_l2(z2, src_b, dst_b)

    out = _tc_fin(agg1, norms_t, b2.reshape(1, _C))
    return out[:_N]


# speedup vs baseline: 8.0328x; 1.0191x over previous
import functools

import jax
import jax.numpy as jnp
from jax import lax
from jax.experimental import pallas as pl
from jax.experimental.pallas import tpu as pltpu
from jax.experimental.pallas import tpu_sc as plsc

_N = 10000
_NPAD = 10240
_E = 160000
_EPAD = 163840
_V = 100000
_H = 256
_HH = 128
_C = 128
_NC = 2
_NS = 16
_NW = _NC * _NS
_RPW = _NPAD // _NW
_ZROWS = _NPAD // _NS

_sc_mesh = functools.partial(
    plsc.VectorSubcoreMesh, core_axis_name="c", subcore_axis_name="s"
)


@functools.partial(
    pl.kernel,
    out_type=(
        jax.ShapeDtypeStruct((_NPAD, _H), jnp.float32),
        jax.ShapeDtypeStruct((_NS, 4, _NPAD), jnp.float32),
    ),
    mesh=_sc_mesh(),
    scratch_types=[
        pltpu.VMEM((4, 80), jnp.int32),
        pltpu.VMEM((4, 80, _H), jnp.float32),
        pltpu.VMEM((80, 128), jnp.int32),
        pltpu.VMEM((_NPAD,), jnp.float32),
        pltpu.VMEM((_NPAD,), jnp.float32),
        pltpu.SemaphoreType.DMA,
        pltpu.SemaphoreType.DMA,
        pltpu.SemaphoreType.DMA,
        pltpu.SemaphoreType.DMA,
        pltpu.SemaphoreType.DMA,
    ],
    compiler_params=pltpu.CompilerParams(needs_layout_passes=False),
)
def _gather_hist(emb, ids, ehist, x_out, degp, ids_v, rows_v, eidx_v,
                 hist_a, hist_b, sg0, sg1, sg2, sg3, sw):
    c = lax.axis_index("c")
    s = lax.axis_index("s")
    wid = c * _NS + s
    gsem = (sg0, sg1, sg2, sg3)

    pltpu.sync_copy(ids.at[wid], ids_v)
    dg = [
        pltpu.async_copy(emb.at[ids_v.at[j]], rows_v.at[j], gsem[j])
        for j in range(4)
    ]

    ones = jnp.ones((16,), jnp.float32)
    zeros = jnp.zeros((16,), jnp.float32)
    for a, href in ((0, hist_a), (1, hist_b)):
        @pl.loop(0, _NPAD // 16)
        def _zero(i):
            href[pl.ds(i * 16, 16)] = zeros

        pltpu.sync_copy(ehist.at[c, a, s], eidx_v)

        @pl.loop(0, 80)
        def _accum(r):
            for jj in range(8):
                plsc.addupdate_scatter(
                    href, [eidx_v[r, pl.ds(jj * 16, 16)]], ones)

        pltpu.sync_copy(href, degp.at[s, c * 2 + a])

    dw = []
    for j in range(4):
        dg[j].wait()
        dw.append(
            pltpu.async_copy(
                rows_v.at[j], x_out.at[pl.ds(wid * _RPW + j * 80, 80)], sw
            )
        )
    for d in dw:
        d.wait()


def _make_spmm(nb, k, table_rows, split_by_core, nseg):
    assert nb % (2 * nseg) == 0
    seg_nb = nb // nseg
    assert seg_nb % 8 == 0

    @functools.partial(
        pl.kernel,
        out_type=jax.ShapeDtypeStruct((_NC, _NPAD, _HH), jnp.float32),
        mesh=_sc_mesh(),
        scratch_types=[
            pltpu.VMEM((seg_nb, k), jnp.int32),
            pltpu.VMEM((seg_nb, k), jnp.int32),
            pltpu.VMEM((k, _HH), jnp.float32),
            pltpu.VMEM((k, _HH), jnp.float32),
            pltpu.VMEM((k, _HH), jnp.float32),
            pltpu.VMEM((k, _HH), jnp.float32),
            pltpu.VMEM_SHARED((_NPAD, _HH), jnp.float32),
            pltpu.SemaphoreType.DMA,
            pltpu.SemaphoreType.DMA,
            pltpu.SemaphoreType.DMA,
            pltpu.SemaphoreType.DMA,
            pltpu.SemaphoreType.DMA,
            pltpu.SemaphoreType.DMA,
            pltpu.SemaphoreType.DMA,
            pltpu.SemaphoreType.DMA,
        ],
        compiler_params=pltpu.CompilerParams(needs_layout_passes=False),
    )
    def spmm(table, srci, dsti, out, idxs_v, idxd_v, b0, b1, b2, b3, acc,
             g0, g1, g2, g3, s0, s1, s2, s3):
        c = lax.axis_index("c")
        s = lax.axis_index("s")
        bufs = (b0, b1, b2, b3)
        gsem = (g0, g1, g2, g3)
        ssem = (s0, s1, s2, s3)

        zeros = jnp.zeros((16,), jnp.float32)

        @pl.loop(0, k)
        def _zrow(i):
            for j in range(_HH // 16):
                b0[i, pl.ds(j * 16, 16)] = zeros

        for t in range(_ZROWS // k):
            pltpu.sync_copy(b0, acc.at[pl.ds(s * _ZROWS + t * k, k)])
        plsc.subcore_barrier()

        def g_start(j, b):
            pltpu.async_copy(table.at[idxs_v.at[j]], bufs[b], gsem[b])

        def g_wait(j, b):
            pltpu.make_async_copy(table.at[idxs_v.at[j]], bufs[b],
                                  gsem[b]).wait()

        def s_start(j, b):
            pltpu.async_copy(bufs[b], acc.at[idxd_v.at[j]], ssem[b],
                             add=True)

        def s_wait(j, b):
            pltpu.make_async_copy(bufs[b], acc.at[idxd_v.at[j]],
                                  ssem[b]).wait()

        for seg in range(nseg):
            lo = seg * seg_nb
            if split_by_core:
                pltpu.sync_copy(srci.at[c, s, pl.ds(lo, seg_nb)], idxs_v)
                pltpu.sync_copy(dsti.at[s, pl.ds(lo, seg_nb)], idxd_v)
            else:
                wid = c * _NS + s
                pltpu.sync_copy(srci.at[wid, pl.ds(lo, seg_nb)], idxs_v)
                pltpu.sync_copy(dsti.at[wid, pl.ds(lo, seg_nb)], idxd_v)

            g_start(0, 0)
            g_start(1, 1)
            g_wait(0, 0); s_start(0, 0); g_start(2, 2)
            g_wait(1, 1); s_start(1, 1); g_start(3, 3)
            g_wait(2, 2); s_start(2, 2); s_wait(0, 0); g_start(4, 0)
            g_wait(3, 3); s_start(3, 3); s_wait(1, 1); g_start(5, 1)

            @pl.loop(1, seg_nb // 4)
            def _steady(jj):
                for b in range(4):
                    j = jj * 4 + b
                    bn = (b + 2) % 4
                    g_wait(j, b)
                    s_start(j, b)
                    s_wait(j - 2, bn)

                    @pl.when(j + 2 < seg_nb)
                    def _():
                        g_start(j + 2, bn)

            s_wait(seg_nb - 2, 2)
            s_wait(seg_nb - 1, 3)

        plsc.subcore_barrier()
        pltpu.sync_copy(acc.at[pl.ds(s * _ZROWS, _ZROWS)],
                        out.at[c, pl.ds(s * _ZROWS, _ZROWS)])

    return spmm


_K1 = 64
_K2 = 64
_spmm_l1 = _make_spmm(_EPAD // _NS // _K1, _K1, _NC * _NPAD, True, 4)
_spmm_l2 = _make_spmm(_EPAD // _NW // _K2, _K2, _NPAD, False, 2)


_BM = 1024


def _scale_body(x_ref, degp_ref, s1_ref, nt_ref):
    d = degp_ref[0]
    for i in range(1, _NS):
        d = d + degp_ref[i]
    nt = lax.rsqrt(jnp.clip(d, 1.0, None)).T
    nt_ref[...] = nt
    t = x_ref[...] * nt[:, 0:1]
    s1_ref[0] = t[:, :_HH]
    s1_ref[1] = t[:, _HH:]


def _mlp_body(a_ref, nt_ref, w1_ref, b1_ref, w2_ref, z_ref):
    h = jnp.concatenate([a_ref[0], a_ref[1]], axis=1) * nt_ref[:, 1:2]
    h = jnp.dot(h, w1_ref[...], preferred_element_type=jnp.float32)
    h = jnp.maximum(h + b1_ref[...], 0.0) * nt_ref[:, 2:3]
    z_ref[...] = jnp.dot(h, w2_ref[...], preferred_element_type=jnp.float32)


def _fin_body(a_ref, nt_ref, b2_ref, o_ref):
    o_ref[...] = (a_ref[0] + a_ref[1]) * nt_ref[:, 3:4] + b2_ref[...]


def _tc_scale(x, degp):
    return pl.pallas_call(
        _scale_body,
        grid=(_NPAD // _BM,),
        in_specs=[
            pl.BlockSpec((_BM, _H), lambda i: (i, 0)),
            pl.BlockSpec((_NS, 4, _BM), lambda i: (0, 0, i)),
        ],
        out_specs=[
            pl.BlockSpec((_NC, _BM, _HH), lambda i: (0, i, 0)),
            pl.BlockSpec((_BM, 4), lambda i: (i, 0)),
        ],
        out_shape=[
            jax.ShapeDtypeStruct((_NC, _NPAD, _HH), jnp.float32),
            jax.ShapeDtypeStruct((_NPAD, 4), jnp.float32),
        ],
    )(x, degp)


def _tc_mlp(agg0, norms_t, w1, b1, w2):
    return pl.pallas_call(
        _mlp_body,
        grid=(_NPAD // _BM,),
        in_specs=[
            pl.BlockSpec((_NC, _BM, _HH), lambda i: (0, i, 0)),
            pl.BlockSpec((_BM, 4), lambda i: (i, 0)),
            pl.BlockSpec((_H, _H), lambda i: (0, 0)),
            pl.BlockSpec((1, _H), lambda i: (0, 0)),
            pl.BlockSpec((_H, _C), lambda i: (0, 0)),
        ],
        out_specs=pl.BlockSpec((_BM, _C), lambda i: (i, 0)),
        out_shape=jax.ShapeDtypeStruct((_NPAD, _C), jnp.float32),
    )(agg0, norms_t, w1, b1, w2)


def _tc_fin(agg1, norms_t, b2):
    return pl.pallas_call(
        _fin_body,
        grid=(_NPAD // _BM,),
        in_specs=[
            pl.BlockSpec((_NC, _BM, _C), lambda i: (0, i, 0)),
            pl.BlockSpec((_BM, 4), lambda i: (i, 0)),
            pl.BlockSpec((1, _C), lambda i: (0, 0)),
        ],
        out_specs=pl.BlockSpec((_BM, _C), lambda i: (i, 0)),
        out_shape=jax.ShapeDtypeStruct((_NPAD, _C), jnp.float32),
    )(agg1, norms_t, b2)


@jax.jit
def kernel(inputs, edge_index0, edge_index1, emb, W1, b1, W2, b2):
    padfill = _N + (jnp.arange(_EPAD - _E, dtype=jnp.int32) % (_NPAD - _N))

    def pad_e(v):
        return jnp.concatenate([v, padfill])

    src0 = pad_e(edge_index0[0])
    dst0 = pad_e(edge_index0[1])
    src1 = pad_e(edge_index1[0])
    dst1 = pad_e(edge_index1[1])

    ids = jnp.concatenate(
        [inputs, jnp.zeros((_NPAD - _N,), jnp.int32)]
    ).reshape(_NW, 4, 80)
    ehist = jnp.stack([src0, dst0, src1, dst1]).reshape(2, 2, _NS, 80, 128)

    x, degp = _gather_hist(emb, ids, ehist)
    s1, norms_t = _tc_scale(x, degp)

    src_a = jnp.stack([src0, src0 + _NPAD]).reshape(_NC, _NS, -1, _K1)
    dst_a = dst0.reshape(_NS, -1, _K1)
    agg0 = _spmm_l1(s1.reshape(_NC * _NPAD, _HH), src_a, dst_a)

    z2 = _tc_mlp(agg0, norms_t, W1, b1.reshape(1, _H), W2)

    src_b = src1.reshape(_NW, -1, _K2)
    dst_b = dst1.reshape(_NW, -1, _K2)
    agg1 = _spmm_l2(z2, src_b, dst_b)

    out = _tc_fin(agg1, norms_t, b2.reshape(1, _C))
    return out[:_N]


# speedup vs baseline: 8.0454x; 1.0016x over previous
import functools

import jax
import jax.numpy as jnp
from jax import lax
from jax.experimental import pallas as pl
from jax.experimental.pallas import tpu as pltpu
from jax.experimental.pallas import tpu_sc as plsc

_N = 10000
_NPAD = 10240
_E = 160000
_EPAD = 163840
_V = 100000
_H = 256
_HH = 128
_C = 128
_NC = 2
_NS = 16
_NW = _NC * _NS
_RPW = _NPAD // _NW
_ZROWS = _NPAD // _NS

_sc_mesh = functools.partial(
    plsc.VectorSubcoreMesh, core_axis_name="c", subcore_axis_name="s"
)


@functools.partial(
    pl.kernel,
    out_type=(
        jax.ShapeDtypeStruct((_NPAD, _H), jnp.float32),
        jax.ShapeDtypeStruct((_NS, 4, _NPAD), jnp.float32),
    ),
    mesh=_sc_mesh(),
    scratch_types=[
        pltpu.VMEM((4, 80), jnp.int32),
        pltpu.VMEM((4, 80, _H), jnp.float32),
        pltpu.VMEM((80, 128), jnp.int32),
        pltpu.VMEM((_NPAD,), jnp.float32),
        pltpu.VMEM((_NPAD,), jnp.float32),
        pltpu.SemaphoreType.DMA,
        pltpu.SemaphoreType.DMA,
        pltpu.SemaphoreType.DMA,
        pltpu.SemaphoreType.DMA,
        pltpu.SemaphoreType.DMA,
    ],
    compiler_params=pltpu.CompilerParams(needs_layout_passes=False),
)
def _gather_hist(emb, ids, ehist, x_out, degp, ids_v, rows_v, eidx_v,
                 hist_a, hist_b, sg0, sg1, sg2, sg3, sw):
    c = lax.axis_index("c")
    s = lax.axis_index("s")
    wid = c * _NS + s
    gsem = (sg0, sg1, sg2, sg3)

    pltpu.sync_copy(ids.at[wid], ids_v)
    dg = [
        pltpu.async_copy(emb.at[ids_v.at[j]], rows_v.at[j], gsem[j])
        for j in range(4)
    ]

    ones = jnp.ones((16,), jnp.float32)
    zeros = jnp.zeros((16,), jnp.float32)
    for a, href in ((0, hist_a), (1, hist_b)):
        @pl.loop(0, _NPAD // 16)
        def _zero(i):
            href[pl.ds(i * 16, 16)] = zeros

        pltpu.sync_copy(ehist.at[c, a, s], eidx_v)

        @pl.loop(0, 80)
        def _accum(r):
            for jj in range(8):
                plsc.addupdate_scatter(
                    href, [eidx_v[r, pl.ds(jj * 16, 16)]], ones)

        pltpu.sync_copy(href, degp.at[s, c * 2 + a])

    dw = []
    for j in range(4):
        dg[j].wait()
        dw.append(
            pltpu.async_copy(
                rows_v.at[j], x_out.at[pl.ds(wid * _RPW + j * 80, 80)], sw
            )
        )
    for d in dw:
        d.wait()


def _make_spmm(nb, k, split_by_core, nseg):
    assert nb % (2 * nseg) == 0
    seg_nb = nb // nseg
    assert seg_nb % 8 == 0

    @functools.partial(
        pl.kernel,
        out_type=jax.ShapeDtypeStruct((_NC, _NPAD, _HH), jnp.float32),
        mesh=_sc_mesh(),
        scratch_types=[
            pltpu.VMEM((seg_nb, k), jnp.int32),
            pltpu.VMEM((seg_nb, k), jnp.int32),
            pltpu.VMEM((k, _HH), jnp.float32),
            pltpu.VMEM((k, _HH), jnp.float32),
            pltpu.VMEM((k, _HH), jnp.float32),
            pltpu.VMEM((k, _HH), jnp.float32),
            pltpu.VMEM_SHARED((_NPAD, _HH), jnp.float32),
            pltpu.SemaphoreType.DMA,
            pltpu.SemaphoreType.DMA,
            pltpu.SemaphoreType.DMA,
            pltpu.SemaphoreType.DMA,
            pltpu.SemaphoreType.DMA,
            pltpu.SemaphoreType.DMA,
            pltpu.SemaphoreType.DMA,
            pltpu.SemaphoreType.DMA,
        ],
        compiler_params=pltpu.CompilerParams(needs_layout_passes=False),
    )
    def spmm(table, srci, dsti, out, idxs_v, idxd_v, b0, b1, b2, b3, acc,
             g0, g1, g2, g3, s0, s1, s2, s3):
        c = lax.axis_index("c")
        s = lax.axis_index("s")
        bufs = (b0, b1, b2, b3)
        gsem = (g0, g1, g2, g3)
        ssem = (s0, s1, s2, s3)

        zeros = jnp.zeros((16,), jnp.float32)

        @pl.loop(0, k)
        def _zrow(i):
            for j in range(_HH // 16):
                b0[i, pl.ds(j * 16, 16)] = zeros

        for t in range(_ZROWS // k):
            pltpu.sync_copy(b0, acc.at[pl.ds(s * _ZROWS + t * k, k)])
        plsc.subcore_barrier()

        def g_start(j, b):
            pltpu.async_copy(table.at[idxs_v.at[j]], bufs[b], gsem[b])

        def g_wait(j, b):
            pltpu.make_async_copy(table.at[idxs_v.at[j]], bufs[b],
                                  gsem[b]).wait()

        def s_start(j, b):
            pltpu.async_copy(bufs[b], acc.at[idxd_v.at[j]], ssem[b],
                             add=True)

        def s_wait(j, b):
            pltpu.make_async_copy(bufs[b], acc.at[idxd_v.at[j]],
                                  ssem[b]).wait()

        for seg in range(nseg):
            lo = seg * seg_nb
            if split_by_core:
                pltpu.sync_copy(srci.at[c, s, pl.ds(lo, seg_nb)], idxs_v)
                pltpu.sync_copy(dsti.at[s, pl.ds(lo, seg_nb)], idxd_v)
            else:
                wid = c * _NS + s
                pltpu.sync_copy(srci.at[wid, pl.ds(lo, seg_nb)], idxs_v)
                pltpu.sync_copy(dsti.at[wid, pl.ds(lo, seg_nb)], idxd_v)

            g_start(0, 0)
            g_start(1, 1)
            g_wait(0, 0); s_start(0, 0); g_start(2, 2)
            g_wait(1, 1); s_start(1, 1); g_start(3, 3)
            g_wait(2, 2); s_start(2, 2); s_wait(0, 0); g_start(4, 0)
            g_wait(3, 3); s_start(3, 3); s_wait(1, 1); g_start(5, 1)

            @pl.loop(1, seg_nb // 4)
            def _steady(jj):
                for b in range(4):
                    j = jj * 4 + b
                    bn = (b + 2) % 4
                    g_wait(j, b)
                    s_start(j, b)
                    s_wait(j - 2, bn)

                    @pl.when(j + 2 < seg_nb)
                    def _():
                        g_start(j + 2, bn)

            s_wait(seg_nb - 2, 2)
            s_wait(seg_nb - 1, 3)

        plsc.subcore_barrier()
        pltpu.sync_copy(acc.at[pl.ds(s * _ZROWS, _ZROWS)],
                        out.at[c, pl.ds(s * _ZROWS, _ZROWS)])

    return spmm


_K1 = 64
_K2 = 64
_spmm_l1 = _make_spmm(_EPAD // _NS // _K1, _K1, True, 4)
_spmm_l2 = _make_spmm(_EPAD // _NW // _K2, _K2, False, 2)


_BM = 1024


def _scale_body(x_ref, degp_ref, s1_ref, nt_ref):
    d = degp_ref[0]
    for i in range(1, _NS):
        d = d + degp_ref[i]
    nt = lax.rsqrt(jnp.clip(d, 1.0, None)).T
    nt_ref[...] = nt
    t = x_ref[...] * nt[:, 0:1]
    s1_ref[0] = t[:, :_HH]
    s1_ref[1] = t[:, _HH:]


def _mlp_body(a_ref, nt_ref, w1_ref, b1_ref, w2_ref, z_ref):
    h = jnp.concatenate([a_ref[0], a_ref[1]], axis=1) * nt_ref[:, 1:2]
    h = jnp.dot(h, w1_ref[...], preferred_element_type=jnp.float32)
    h = jnp.maximum(h + b1_ref[...], 0.0) * nt_ref[:, 2:3]
    z_ref[...] = jnp.dot(h, w2_ref[...], preferred_element_type=jnp.float32)


def _fin_body(a_ref, nt_ref, b2_ref, o_ref):
    o_ref[...] = (a_ref[0] + a_ref[1]) * nt_ref[:, 3:4] + b2_ref[...]


def _tc_scale(x, degp):
    return pl.pallas_call(
        _scale_body,
        grid=(_NPAD // _BM,),
        in_specs=[
            pl.BlockSpec((_BM, _H), lambda i: (i, 0)),
            pl.BlockSpec((_NS, 4, _BM), lambda i: (0, 0, i)),
        ],
        out_specs=[
            pl.BlockSpec((_NC, _BM, _HH), lambda i: (0, i, 0)),
            pl.BlockSpec((_BM, 4), lambda i: (i, 0)),
        ],
        out_shape=[
            jax.ShapeDtypeStruct((_NC, _NPAD, _HH), jnp.float32),
            jax.ShapeDtypeStruct((_NPAD, 4), jnp.float32),
        ],
    )(x, degp)


def _tc_mlp(agg0, norms_t, w1, b1, w2):
    return pl.pallas_call(
        _mlp_body,
        grid=(_NPAD // _BM,),
        in_specs=[
            pl.BlockSpec((_NC, _BM, _HH), lambda i: (0, i, 0)),
            pl.BlockSpec((_BM, 4), lambda i: (i, 0)),
            pl.BlockSpec((_H, _H), lambda i: (0, 0)),
            pl.BlockSpec((1, _H), lambda i: (0, 0)),
            pl.BlockSpec((_H, _C), lambda i: (0, 0)),
        ],
        out_specs=pl.BlockSpec((_BM, _C), lambda i: (i, 0)),
        out_shape=jax.ShapeDtypeStruct((_NPAD, _C), jnp.float32),
    )(agg0, norms_t, w1, b1, w2)


def _tc_fin(agg1, norms_t, b2):
    return pl.pallas_call(
        _fin_body,
        grid=(_NPAD // _BM,),
        in_specs=[
            pl.BlockSpec((_NC, _BM, _C), lambda i: (0, i, 0)),
            pl.BlockSpec((_BM, 4), lambda i: (i, 0)),
            pl.BlockSpec((1, _C), lambda i: (0, 0)),
        ],
        out_specs=pl.BlockSpec((_BM, _C), lambda i: (i, 0)),
        out_shape=jax.ShapeDtypeStruct((_NPAD, _C), jnp.float32),
    )(agg1, norms_t, b2)


@jax.jit
def kernel(inputs, edge_index0, edge_index1, emb, W1, b1, W2, b2):
    padfill = _N + (jnp.arange(_EPAD - _E, dtype=jnp.int32) % (_NPAD - _N))

    def pad_e(v):
        return jnp.concatenate([v, padfill])

    src0 = pad_e(edge_index0[0])
    dst0 = pad_e(edge_index0[1])
    src1 = pad_e(edge_index1[0])
    dst1 = pad_e(edge_index1[1])

    ids = jnp.concatenate(
        [inputs, jnp.zeros((_NPAD - _N,), jnp.int32)]
    ).reshape(_NW, 4, 80)
    ehist = jnp.stack([src0, dst0, src1, dst1]).reshape(2, 2, _NS, 80, 128)

    x, degp = _gather_hist(emb, ids, ehist)
    s1, norms_t = _tc_scale(x, degp)

    src_a = jnp.stack([src0, src0 + _NPAD]).reshape(_NC, _NS, -1, _K1)
    dst_a = dst0.reshape(_NS, -1, _K1)
    agg0 = _spmm_l1(s1.reshape(_NC * _NPAD, _HH), src_a, dst_a)

    z2 = _tc_mlp(agg0, norms_t, W1, b1.reshape(1, _H), W2)

    src_b = src1.reshape(_NW, -1, _K2)
    dst_b = dst1.reshape(_NW, -1, _K2)
    agg1 = _spmm_l2(z2, src_b, dst_b)

    out = _tc_fin(agg1, norms_t, b2.reshape(1, _C))
    return out[:_N]
